# baseline (speedup 1.0000x reference)
import jax
import jax.numpy as jnp
from jax import lax
from jax.experimental import pallas as pl
from jax.experimental.pallas import tpu as pltpu

_BB = 4
_D = 6


def kernel(x, removed_electrodes):
    B, C, T, E = x.shape
    xt = jnp.transpose(x, (0, 3, 1, 2))
    NB = B // _BB
    TOT = E * NB
    rem = removed_electrodes.astype(jnp.int32)
    zeros = jnp.zeros((_BB, C, T), x.dtype)

    def body(rem_ref, z_ref, x_hbm, o_hbm, sems):
        s = pl.program_id(0)

        def keep_of(t):
            e_t = t // NB
            k = jnp.int32(1)
            for j in range(rem_ref.shape[0]):
                k = k * (e_t + 1 != rem_ref[j]).astype(jnp.int32)
            return k

        def slices(t):
            e_t = t // NB
            j_t = lax.rem(t, NB)
            return pl.ds(j_t * _BB, _BB), e_t

        def issue(t):
            bsl, e_t = slices(t)
            kp = keep_of(t)
            sem = sems.at[lax.rem(t, _D)]

            @pl.when(kp == 1)
            def _():
                pltpu.make_async_copy(
                    x_hbm.at[bsl, e_t], o_hbm.at[bsl, e_t], sem
                ).start()

            @pl.when(kp == 0)
            def _():
                pltpu.make_async_copy(z_ref, o_hbm.at[bsl, e_t], sem).start()

        def wait(t):
            bsl, e_t = slices(t)
            pltpu.make_async_copy(
                z_ref, o_hbm.at[bsl, e_t], sems.at[lax.rem(t, _D)]
            ).wait()

        issue(s)

        @pl.when(s >= _D)
        def _():
            wait(s - _D)

        @pl.when(s == TOT - 1)
        def _():
            for d in range(_D):
                wait(s - d)

    out_t = pl.pallas_call(
        body,
        grid=(TOT,),
        in_specs=[
            pl.BlockSpec(memory_space=pltpu.SMEM),
            pl.BlockSpec(memory_space=pltpu.VMEM),
            pl.BlockSpec(memory_space=pl.ANY),
        ],
        out_specs=pl.BlockSpec(memory_space=pl.ANY),
        out_shape=jax.ShapeDtypeStruct((B, E, C, T), x.dtype),
        scratch_shapes=[pltpu.SemaphoreType.DMA((_D,))],
    )(rem, zeros, xt)
    return jnp.transpose(out_t, (0, 2, 3, 1))


# speedup vs baseline: 25.6718x; 25.6718x over previous
import jax
import jax.numpy as jnp
from jax import lax
from jax.experimental import pallas as pl
from jax.experimental.pallas import tpu as pltpu

_BB = 8
_NBUF = 3


def kernel(x, removed_electrodes):
    B, C, T, E = x.shape
    xt = jnp.transpose(x, (0, 3, 1, 2))
    NB = B // _BB
    TOT = E * NB
    rem = removed_electrodes.astype(jnp.int32)

    def body(rem_ref, x_hbm, o_ref, buf, sems):
        e = pl.program_id(0)
        j = pl.program_id(1)
        s = e * NB + j

        def keep_of(t):
            e_t = t // NB
            k = jnp.int32(1)
            for i in range(rem_ref.shape[0]):
                k = k * (e_t + 1 != rem_ref[i]).astype(jnp.int32)
            return k

        def issue(t):
            @pl.when(keep_of(t) == 1)
            def _():
                e_t = t // NB
                j_t = lax.rem(t, NB)
                slot = lax.rem(t, _NBUF)
                pltpu.make_async_copy(
                    x_hbm.at[pl.ds(j_t * _BB, _BB), e_t],
                    buf.at[slot],
                    sems.at[slot],
                ).start()

        @pl.when(s == 0)
        def _():
            issue(0)

        @pl.when(s + 1 < TOT)
        def _():
            issue(s + 1)

        kp = keep_of(s)

        @pl.when(kp == 1)
        def _():
            slot = lax.rem(s, _NBUF)
            pltpu.make_async_copy(
                x_hbm.at[pl.ds(0, _BB), 0], buf.at[slot], sems.at[slot]
            ).wait()
            o_ref[:, 0] = buf[slot]

        @pl.when(kp == 0)
        def _():
            o_ref[...] = jnp.zeros_like(o_ref)

    out_t = pl.pallas_call(
        body,
        grid=(E, NB),
        in_specs=[
            pl.BlockSpec(memory_space=pltpu.SMEM),
            pl.BlockSpec(memory_space=pl.ANY),
        ],
        out_specs=pl.BlockSpec((_BB, 1, C, T), lambda e, j: (j, e, 0, 0)),
        out_shape=jax.ShapeDtypeStruct((B, E, C, T), x.dtype),
        scratch_shapes=[
            pltpu.VMEM((_NBUF, _BB, C, T), x.dtype),
            pltpu.SemaphoreType.DMA((_NBUF,)),
        ],
    )(rem, xt)
    return jnp.transpose(out_t, (0, 2, 3, 1))


# speedup vs baseline: 33.8901x; 1.3201x over previous
import jax
import jax.numpy as jnp
from jax import lax
from jax.experimental import pallas as pl
from jax.experimental.pallas import tpu as pltpu

_BB = 8
_NBUF = 4
_LA = 2


def kernel(x, removed_electrodes):
    B, C, T, E = x.shape
    xt = jnp.transpose(x, (0, 3, 1, 2))
    NB = B // _BB
    TOT = E * NB
    rem = removed_electrodes.astype(jnp.int32)
    zeros = jnp.zeros((_BB, C, T), x.dtype)

    def body(rem_ref, z_ref, x_hbm, o_hbm, buf, in_sems, out_sems):
        e = pl.program_id(0)
        j = pl.program_id(1)
        s = e * NB + j

        def keep_of(t):
            e_t = t // NB
            k = jnp.int32(1)
            for i in range(rem_ref.shape[0]):
                k = k * (e_t + 1 != rem_ref[i]).astype(jnp.int32)
            return k

        def slices(t):
            e_t = t // NB
            j_t = lax.rem(t, NB)
            return pl.ds(j_t * _BB, _BB), e_t

        def issue_in(t):
            @pl.when(keep_of(t) == 1)
            def _():
                bsl, e_t = slices(t)
                slot = lax.rem(t, _NBUF)
                pltpu.make_async_copy(
                    x_hbm.at[bsl, e_t], buf.at[slot], in_sems.at[slot]
                ).start()

        def wait_in(t):
            slot = lax.rem(t, _NBUF)
            pltpu.make_async_copy(
                x_hbm.at[pl.ds(0, _BB), 0], buf.at[slot], in_sems.at[slot]
            ).wait()

        def issue_out(t):
            bsl, e_t = slices(t)
            slot = lax.rem(t, _NBUF)
            kp = keep_of(t)

            @pl.when(kp == 1)
            def _():
                pltpu.make_async_copy(
                    buf.at[slot], o_hbm.at[bsl, e_t], out_sems.at[slot]
                ).start()

            @pl.when(kp == 0)
            def _():
                pltpu.make_async_copy(
                    z_ref, o_hbm.at[bsl, e_t], out_sems.at[slot]
                ).start()

        def wait_out(t):
            bsl, e_t = slices(t)
            pltpu.make_async_copy(
                z_ref, o_hbm.at[bsl, e_t], out_sems.at[lax.rem(t, _NBUF)]
            ).wait()

        @pl.when(s == 0)
        def _():
            for t0 in range(_LA):
                issue_in(jnp.int32(t0))

        @pl.when(s >= _LA)
        def _():
            wait_out(s - _LA)

        @pl.when(s + _LA < TOT)
        def _():
            issue_in(s + _LA)

        @pl.when(keep_of(s) == 1)
        def _():
            wait_in(s)

        issue_out(s)

        @pl.when(s == TOT - 1)
        def _():
            for d in range(_LA):
                wait_out(s - d)

    out_t = pl.pallas_call(
        body,
        grid=(E, NB),
        in_specs=[
            pl.BlockSpec(memory_space=pltpu.SMEM),
            pl.BlockSpec(memory_space=pltpu.VMEM),
            pl.BlockSpec(memory_space=pl.ANY),
        ],
        out_specs=pl.BlockSpec(memory_space=pl.ANY),
        out_shape=jax.ShapeDtypeStruct((B, E, C, T), x.dtype),
        scratch_shapes=[
            pltpu.VMEM((_NBUF, _BB, C, T), x.dtype),
            pltpu.SemaphoreType.DMA((_NBUF,)),
            pltpu.SemaphoreType.DMA((_NBUF,)),
        ],
    )(rem, zeros, xt)
    return jnp.transpose(out_t, (0, 2, 3, 1))


# speedup vs baseline: 35.4359x; 1.0456x over previous
import jax
import jax.numpy as jnp
from jax import lax
from jax.experimental import pallas as pl
from jax.experimental.pallas import tpu as pltpu

_BB = 4
_NBUF = 8
_LA = 4


def kernel(x, removed_electrodes):
    B, C, T, E = x.shape
    xt = jnp.transpose(x, (0, 3, 1, 2))
    NB = B // _BB
    TOT = E * NB
    rem = removed_electrodes.astype(jnp.int32)
    zeros = jnp.zeros((_BB, C, T), x.dtype)

    def body(rem_ref, z_ref, x_hbm, o_hbm, buf, in_sems, out_sems):
        e = pl.program_id(0)
        j = pl.program_id(1)
        s = e * NB + j

        def keep_of(t):
            e_t = t // NB
            k = jnp.int32(1)
            for i in range(rem_ref.shape[0]):
                k = k * (e_t + 1 != rem_ref[i]).astype(jnp.int32)
            return k

        def slices(t):
            e_t = t // NB
            j_t = lax.rem(t, NB)
            return pl.ds(j_t * _BB, _BB), e_t

        def issue_in(t):
            @pl.when(keep_of(t) == 1)
            def _():
                bsl, e_t = slices(t)
                slot = lax.rem(t, _NBUF)
                pltpu.make_async_copy(
                    x_hbm.at[bsl, e_t], buf.at[slot], in_sems.at[slot]
                ).start()

        def wait_in(t):
            slot = lax.rem(t, _NBUF)
            pltpu.make_async_copy(
                x_hbm.at[pl.ds(0, _BB), 0], buf.at[slot], in_sems.at[slot]
            ).wait()

        def issue_out(t):
            bsl, e_t = slices(t)
            slot = lax.rem(t, _NBUF)
            kp = keep_of(t)

            @pl.when(kp == 1)
            def _():
                pltpu.make_async_copy(
                    buf.at[slot], o_hbm.at[bsl, e_t], out_sems.at[slot]
                ).start()

            @pl.when(kp == 0)
            def _():
                pltpu.make_async_copy(
                    z_ref, o_hbm.at[bsl, e_t], out_sems.at[slot]
                ).start()

        def wait_out(t):
            bsl, e_t = slices(t)
            pltpu.make_async_copy(
                z_ref, o_hbm.at[bsl, e_t], out_sems.at[lax.rem(t, _NBUF)]
            ).wait()

        @pl.when(s == 0)
        def _():
            for t0 in range(_LA):
                issue_in(jnp.int32(t0))

        @pl.when(s >= _LA)
        def _():
            wait_out(s - _LA)

        @pl.when(s + _LA < TOT)
        def _():
            issue_in(s + _LA)

        @pl.when(keep_of(s) == 1)
        def _():
            wait_in(s)

        issue_out(s)

        @pl.when(s == TOT - 1)
        def _():
            for d in range(_LA):
                wait_out(s - d)

    out_t = pl.pallas_call(
        body,
        grid=(E, NB),
        in_specs=[
            pl.BlockSpec(memory_space=pltpu.SMEM),
            pl.BlockSpec(memory_space=pltpu.VMEM),
            pl.BlockSpec(memory_space=pl.ANY),
        ],
        out_specs=pl.BlockSpec(memory_space=pl.ANY),
        out_shape=jax.ShapeDtypeStruct((B, E, C, T), x.dtype),
        scratch_shapes=[
            pltpu.VMEM((_NBUF, _BB, C, T), x.dtype),
            pltpu.SemaphoreType.DMA((_NBUF,)),
            pltpu.SemaphoreType.DMA((_NBUF,)),
        ],
    )(rem, zeros, xt)
    return jnp.transpose(out_t, (0, 2, 3, 1))


# speedup vs baseline: 39.8696x; 1.1251x over previous
import jax
import jax.numpy as jnp
from jax import lax
from jax.experimental import pallas as pl
from jax.experimental.pallas import tpu as pltpu

_BB = 4
_NBUF = 16
_LA = 8


def kernel(x, removed_electrodes):
    B, C, T, E = x.shape
    xt = jnp.transpose(x, (0, 3, 1, 2))
    NB = B // _BB
    TOT = E * NB
    rem = removed_electrodes.astype(jnp.int32)
    zeros = jnp.zeros((_BB, C, T), x.dtype)

    def body(rem_ref, z_ref, x_hbm, o_hbm, buf, in_sems, out_sems):
        e = pl.program_id(0)
        j = pl.program_id(1)
        s = e * NB + j

        def keep_of(t):
            e_t = t // NB
            k = jnp.int32(1)
            for i in range(rem_ref.shape[0]):
                k = k * (e_t + 1 != rem_ref[i]).astype(jnp.int32)
            return k

        def slices(t):
            e_t = t // NB
            j_t = lax.rem(t, NB)
            return pl.ds(j_t * _BB, _BB), e_t

        def issue_in(t):
            @pl.when(keep_of(t) == 1)
            def _():
                bsl, e_t = slices(t)
                slot = lax.rem(t, _NBUF)
                pltpu.make_async_copy(
                    x_hbm.at[bsl, e_t], buf.at[slot], in_sems.at[slot]
                ).start()

        def wait_in(t):
            slot = lax.rem(t, _NBUF)
            pltpu.make_async_copy(
                x_hbm.at[pl.ds(0, _BB), 0], buf.at[slot], in_sems.at[slot]
            ).wait()

        def issue_out(t):
            bsl, e_t = slices(t)
            slot = lax.rem(t, _NBUF)
            kp = keep_of(t)

            @pl.when(kp == 1)
            def _():
                pltpu.make_async_copy(
                    buf.at[slot], o_hbm.at[bsl, e_t], out_sems.at[slot]
                ).start()

            @pl.when(kp == 0)
            def _():
                pltpu.make_async_copy(
                    z_ref, o_hbm.at[bsl, e_t], out_sems.at[slot]
                ).start()

        def wait_out(t):
            bsl, e_t = slices(t)
            pltpu.make_async_copy(
                z_ref, o_hbm.at[bsl, e_t], out_sems.at[lax.rem(t, _NBUF)]
            ).wait()

        @pl.when(s == 0)
        def _():
            for t0 in range(_LA):
                issue_in(jnp.int32(t0))

        @pl.when(s >= _LA)
        def _():
            wait_out(s - _LA)

        @pl.when(s + _LA < TOT)
        def _():
            issue_in(s + _LA)

        @pl.when(keep_of(s) == 1)
        def _():
            wait_in(s)

        issue_out(s)

        @pl.when(s == TOT - 1)
        def _():
            for d in range(_LA):
                wait_out(s - d)

    out_t = pl.pallas_call(
        body,
        grid=(E, NB),
        in_specs=[
            pl.BlockSpec(memory_space=pltpu.SMEM),
            pl.BlockSpec(memory_space=pltpu.VMEM),
            pl.BlockSpec(memory_space=pl.ANY),
        ],
        out_specs=pl.BlockSpec(memory_space=pl.ANY),
        out_shape=jax.ShapeDtypeStruct((B, E, C, T), x.dtype),
        scratch_shapes=[
            pltpu.VMEM((_NBUF, _BB, C, T), x.dtype),
            pltpu.SemaphoreType.DMA((_NBUF,)),
            pltpu.SemaphoreType.DMA((_NBUF,)),
        ],
    )(rem, zeros, xt)
    return jnp.transpose(out_t, (0, 2, 3, 1))


# speedup vs baseline: 40.2064x; 1.0084x over previous
import jax
import jax.numpy as jnp
from jax import lax
from jax.experimental import pallas as pl
from jax.experimental.pallas import tpu as pltpu

_BB = 4
_NBUF = 24
_LA = 12


def kernel(x, removed_electrodes):
    B, C, T, E = x.shape
    xt = jnp.transpose(x, (0, 3, 1, 2))
    NB = B // _BB
    TOT = E * NB
    rem = removed_electrodes.astype(jnp.int32)
    zeros = jnp.zeros((_BB, C, T), x.dtype)

    def body(rem_ref, z_ref, x_hbm, o_hbm, buf, in_sems, out_sems):
        e = pl.program_id(0)
        j = pl.program_id(1)
        s = e * NB + j

        def keep_of(t):
            e_t = t // NB
            k = jnp.int32(1)
            for i in range(rem_ref.shape[0]):
                k = k * (e_t + 1 != rem_ref[i]).astype(jnp.int32)
            return k

        def slices(t):
            e_t = t // NB
            j_t = lax.rem(t, NB)
            return pl.ds(j_t * _BB, _BB), e_t

        def issue_in(t):
            @pl.when(keep_of(t) == 1)
            def _():
                bsl, e_t = slices(t)
                slot = lax.rem(t, _NBUF)
                pltpu.make_async_copy(
                    x_hbm.at[bsl, e_t], buf.at[slot], in_sems.at[slot]
                ).start()

        def wait_in(t):
            slot = lax.rem(t, _NBUF)
            pltpu.make_async_copy(
                x_hbm.at[pl.ds(0, _BB), 0], buf.at[slot], in_sems.at[slot]
            ).wait()

        def issue_out(t):
            bsl, e_t = slices(t)
            slot = lax.rem(t, _NBUF)
            kp = keep_of(t)

            @pl.when(kp == 1)
            def _():
                pltpu.make_async_copy(
                    buf.at[slot], o_hbm.at[bsl, e_t], out_sems.at[slot]
                ).start()

            @pl.when(kp == 0)
            def _():
                pltpu.make_async_copy(
                    z_ref, o_hbm.at[bsl, e_t], out_sems.at[slot]
                ).start()

        def wait_out(t):
            bsl, e_t = slices(t)
            pltpu.make_async_copy(
                z_ref, o_hbm.at[bsl, e_t], out_sems.at[lax.rem(t, _NBUF)]
            ).wait()

        @pl.when(s == 0)
        def _():
            for t0 in range(_LA):
                issue_in(jnp.int32(t0))

        @pl.when(s >= _LA)
        def _():
            wait_out(s - _LA)

        @pl.when(s + _LA < TOT)
        def _():
            issue_in(s + _LA)

        @pl.when(keep_of(s) == 1)
        def _():
            wait_in(s)

        issue_out(s)

        @pl.when(s == TOT - 1)
        def _():
            for d in range(_LA):
                wait_out(s - d)

    out_t = pl.pallas_call(
        body,
        grid=(E, NB),
        in_specs=[
            pl.BlockSpec(memory_space=pltpu.SMEM),
            pl.BlockSpec(memory_space=pltpu.VMEM),
            pl.BlockSpec(memory_space=pl.ANY),
        ],
        out_specs=pl.BlockSpec(memory_space=pl.ANY),
        out_shape=jax.ShapeDtypeStruct((B, E, C, T), x.dtype),
        scratch_shapes=[
            pltpu.VMEM((_NBUF, _BB, C, T), x.dtype),
            pltpu.SemaphoreType.DMA((_NBUF,)),
            pltpu.SemaphoreType.DMA((_NBUF,)),
        ],
    )(rem, zeros, xt)
    return jnp.transpose(out_t, (0, 2, 3, 1))
